# baseline (speedup 1.0000x reference)
import jax
import jax.numpy as jnp
from jax.experimental import pallas as pl
from jax.experimental.pallas import tpu as pltpu

HIDDEN = 2048
N_EXPERTS = 16
TOP_K = 2
TILE = 1024
NBUF = 4


def _router_body(x_hbm, w_ref, w_out_ref, tw_out_ref, te_out_ref, xbuf, sems):
    n_steps = x_hbm.shape[0] // TILE

    def start_copy(step):
        slot = jax.lax.rem(step, NBUF)
        pltpu.make_async_copy(
            x_hbm.at[pl.ds(step * TILE, TILE), :],
            xbuf.at[slot],
            sems.at[slot],
        ).start()

    for k in range(NBUF - 1):
        start_copy(k)

    def loop(step, carry):
        slot = jax.lax.rem(step, NBUF)
        pltpu.make_async_copy(
            x_hbm.at[pl.ds(step * TILE, TILE), :],
            xbuf.at[slot],
            sems.at[slot],
        ).wait()

        nxt = step + NBUF - 1

        @pl.when(nxt < n_steps)
        def _():
            start_copy(nxt)

        lt = jax.lax.dot_general(
            w_ref[...], xbuf[slot],
            dimension_numbers=(((1,), (1,)), ((), ())),
            preferred_element_type=jnp.float32,
        )
        m = jnp.max(lt, axis=0, keepdims=True)
        e = jnp.exp(lt - m)
        s = jnp.sum(e, axis=0, keepdims=True)
        r = 1.0 / s
        row = step * TILE
        w_out_ref[pl.ds(row, TILE), :] = (e * r).T

        iota = jax.lax.broadcasted_iota(jnp.int32, lt.shape, 0)
        i1 = jnp.min(jnp.where(lt == m, iota, N_EXPERTS), axis=0, keepdims=True)
        masked = jnp.where(iota == i1, -jnp.inf, lt)
        m2 = jnp.max(masked, axis=0, keepdims=True)
        i2 = jnp.min(jnp.where(masked == m2, iota, N_EXPERTS), axis=0, keepdims=True)
        tw = jnp.concatenate([r, jnp.exp(m2 - m) * r], axis=0)
        te = jnp.concatenate([i1, i2], axis=0)
        tw_out_ref[pl.ds(row, TILE), :] = tw.T
        te_out_ref[pl.ds(row, TILE), :] = te.T
        return carry

    jax.lax.fori_loop(0, n_steps, loop, 0)


def kernel(x, W):
    n = x.shape[0]
    weights, top_w, top_e = pl.pallas_call(
        _router_body,
        in_specs=[
            pl.BlockSpec(memory_space=pl.ANY),
            pl.BlockSpec(memory_space=pltpu.MemorySpace.VMEM),
        ],
        out_specs=[
            pl.BlockSpec(memory_space=pltpu.MemorySpace.VMEM),
            pl.BlockSpec(memory_space=pltpu.MemorySpace.VMEM),
            pl.BlockSpec(memory_space=pltpu.MemorySpace.VMEM),
        ],
        out_shape=[
            jax.ShapeDtypeStruct((n, N_EXPERTS), jnp.float32),
            jax.ShapeDtypeStruct((n, TOP_K), jnp.float32),
            jax.ShapeDtypeStruct((n, TOP_K), jnp.int32),
        ],
        scratch_shapes=[
            pltpu.VMEM((NBUF, TILE, HIDDEN), jnp.float32),
            pltpu.SemaphoreType.DMA((NBUF,)),
        ],
    )(x, W)
    return (weights, top_w, top_e.astype(jnp.int64))


# speedup vs baseline: 1.6525x; 1.6525x over previous
import jax
import jax.numpy as jnp
from jax.experimental import pallas as pl
from jax.experimental.pallas import tpu as pltpu

HIDDEN = 2048
N_EXPERTS = 16
TOP_K = 2
TILE = 1024


def _router_body(x_ref, w_ref, w_out_ref, tw_out_ref, te_out_ref):
    lt = jax.lax.dot_general(
        w_ref[...], x_ref[...],
        dimension_numbers=(((1,), (1,)), ((), ())),
        preferred_element_type=jnp.float32,
    )
    m = jnp.max(lt, axis=0, keepdims=True)
    e = jnp.exp(lt - m)
    s = jnp.sum(e, axis=0, keepdims=True)
    r = 1.0 / s
    w_out_ref[...] = e * r

    iota = jax.lax.broadcasted_iota(jnp.int32, lt.shape, 0)
    i1 = jnp.min(jnp.where(lt == m, iota, N_EXPERTS), axis=0, keepdims=True)
    masked = jnp.where(iota == i1, -jnp.inf, lt)
    m2 = jnp.max(masked, axis=0, keepdims=True)
    i2 = jnp.min(jnp.where(masked == m2, iota, N_EXPERTS), axis=0, keepdims=True)
    tw_out_ref[...] = jnp.concatenate([r, jnp.exp(m2 - m) * r], axis=0)
    te_out_ref[...] = jnp.concatenate([i1, i2], axis=0)


def kernel(x, W):
    n = x.shape[0]
    grid = (n // TILE,)
    weights_t, top_w_t, top_e_t = pl.pallas_call(
        _router_body,
        grid=grid,
        in_specs=[
            pl.BlockSpec((TILE, HIDDEN), lambda i: (i, 0)),
            pl.BlockSpec((N_EXPERTS, HIDDEN), lambda i: (0, 0)),
        ],
        out_specs=[
            pl.BlockSpec((N_EXPERTS, TILE), lambda i: (0, i)),
            pl.BlockSpec((TOP_K, TILE), lambda i: (0, i)),
            pl.BlockSpec((TOP_K, TILE), lambda i: (0, i)),
        ],
        out_shape=[
            jax.ShapeDtypeStruct((N_EXPERTS, n), jnp.float32),
            jax.ShapeDtypeStruct((TOP_K, n), jnp.float32),
            jax.ShapeDtypeStruct((TOP_K, n), jnp.int32),
        ],
        compiler_params=pltpu.CompilerParams(
            dimension_semantics=("parallel",),
        ),
    )(x, W)
    return (
        weights_t.T,
        top_w_t.T,
        top_e_t.T.astype(jnp.int64),
    )
